# baseline (speedup 1.0000x reference)
import functools

import jax
import jax.numpy as jnp
from jax import lax
from jax.experimental import pallas as pl
from jax.experimental.pallas import tpu as pltpu
from jax.experimental.pallas import tpu_sc as plsc

_B, _S, _D = 4, 8192, 1024
_NC, _NS = 2, 16
_NW = _NC * _NS
_RB = 8
_NB = _S // _RB
_PB = _NB // _NW
_CB = 2
_BB = 2
_NCH = (_PB // _CB) * (_B // _BB)
_GPR = _D // 16
_LANES = 16
_DEPTH = 2
_UNROLL = 8
_SUBR = _GPR // _UNROLL


def _add_chunk(xbuf, pbuf):
    def body(i, carry):
        cr = i // _SUBR
        j = (i % _SUBR) * _UNROLL
        c = cr // _RB
        r = cr % _RB
        for u in range(_UNROLL):
            c16 = (j + u) * _LANES
            pv = pbuf[c, r, pl.ds(c16, _LANES)]
            for b in range(_BB):
                xbuf[b, c, r, pl.ds(c16, _LANES)] = (
                    xbuf[b, c, r, pl.ds(c16, _LANES)] + pv)
        return carry
    lax.fori_loop(0, _CB * _RB * _SUBR, body, 0)


def _sc_body(x_hbm, pos_hbm, out_hbm, *scratch):
    pbufs = scratch[0:_DEPTH]
    xbufs = scratch[_DEPTH:2 * _DEPTH]
    ldsems = scratch[2 * _DEPTH:3 * _DEPTH]
    stsems = scratch[3 * _DEPTH:4 * _DEPTH]
    wid = lax.axis_index("s") * _NC + lax.axis_index("c")
    band_base = wid * _PB

    def chunk_coords(ch):
        bp = ch % (_B // _BB)
        bc = ch // (_B // _BB)
        return bp * _BB, band_base + bc * _CB

    def issue_loads(ch, slot):
        b0, p0 = chunk_coords(ch)
        descs = [pltpu.make_async_copy(
            pos_hbm.at[pl.ds(p0, _CB)], pbufs[slot], ldsems[slot])]
        for b in range(_BB):
            descs.append(pltpu.make_async_copy(
                x_hbm.at[pl.ds((b0 + b) * _NB + p0, _CB)],
                xbufs[slot].at[b], ldsems[slot]))
        for d in descs:
            d.start()
        return descs

    def issue_stores(ch, slot):
        b0, p0 = chunk_coords(ch)
        descs = []
        for b in range(_BB):
            descs.append(pltpu.make_async_copy(
                xbufs[slot].at[b],
                out_hbm.at[pl.ds((b0 + b) * _NB + p0, _CB)], stsems[slot]))
        for d in descs:
            d.start()
        return descs

    loads = {}
    stores = {}
    for c in range(min(_DEPTH - 1, _NCH)):
        loads[c] = issue_loads(c, c % _DEPTH)
    for c in range(_NCH):
        slot = c % _DEPTH
        if c + _DEPTH - 1 < _NCH:
            nslot = (c + _DEPTH - 1) % _DEPTH
            if c - 1 >= 0:
                for d in stores[c - 1]:
                    d.wait()
            loads[c + _DEPTH - 1] = issue_loads(c + _DEPTH - 1, nslot)
        for d in loads[c]:
            d.wait()
        _add_chunk(xbufs[slot], pbufs[slot])
        stores[c] = issue_stores(c, slot)
    for c in range(max(0, _NCH - _DEPTH), _NCH):
        for d in stores[c]:
            d.wait()


def kernel(x, pos_table):
    batch, seq_len, d_model = x.shape
    xv = x.reshape(batch * seq_len // _RB, _RB, d_model)
    pv = pos_table.reshape(seq_len // _RB, _RB, d_model)
    run = functools.partial(
        pl.kernel,
        out_type=jax.ShapeDtypeStruct(xv.shape, x.dtype),
        scratch_types=(
            [pltpu.VMEM((_CB, _RB, _D), jnp.float32)] * _DEPTH
            + [pltpu.VMEM((_BB, _CB, _RB, _D), jnp.float32)] * _DEPTH
            + [pltpu.SemaphoreType.DMA] * (2 * _DEPTH)
        ),
        mesh=plsc.VectorSubcoreMesh(core_axis_name="c", subcore_axis_name="s"),
    )(_sc_body)
    out = run(xv, pv)
    return out.reshape(batch, seq_len, d_model)


# speedup vs baseline: 1.0411x; 1.0411x over previous
import functools

import jax
import jax.numpy as jnp
from jax import lax
from jax.experimental import pallas as pl
from jax.experimental.pallas import tpu as pltpu
from jax.experimental.pallas import tpu_sc as plsc

_B, _S, _D = 4, 8192, 1024
_NC, _NS = 2, 16
_NW = _NC * _NS
_RB = 8
_NB = _S // _RB
_PB = _NB // _NW
_GPR = _D // 16
_LANES = 16
_DEPTH = 3


_UNROLL = 8
_SUBR = _GPR // _UNROLL


def _add_chunk(xbuf, pbuf):
    def body(i, carry):
        r = i // _SUBR
        j = (i % _SUBR) * _UNROLL
        for u in range(_UNROLL):
            c16 = (j + u) * _LANES
            pv = pbuf[r, pl.ds(c16, _LANES)]
            for b in range(_B):
                xbuf[b, r, pl.ds(c16, _LANES)] = (
                    xbuf[b, r, pl.ds(c16, _LANES)] + pv)
        return carry
    lax.fori_loop(0, _RB * _SUBR, body, 0)


def _sc_body(x_hbm, pos_hbm, out_hbm, *scratch):
    pbufs = scratch[0:_DEPTH]
    xbufs = scratch[_DEPTH:2 * _DEPTH]
    ldsems = scratch[2 * _DEPTH:3 * _DEPTH]
    stsems = scratch[3 * _DEPTH:4 * _DEPTH]
    wid = lax.axis_index("s") * _NC + lax.axis_index("c")
    band_base = wid * _PB

    def issue_loads(c, slot):
        p0 = band_base + c
        descs = [pltpu.make_async_copy(
            pos_hbm.at[p0], pbufs[slot], ldsems[slot])]
        for b in range(_B):
            descs.append(pltpu.make_async_copy(
                x_hbm.at[b * _NB + p0], xbufs[slot].at[b], ldsems[slot]))
        for d in descs:
            d.start()
        return descs

    def issue_stores(c, slot):
        p0 = band_base + c
        descs = []
        for b in range(_B):
            descs.append(pltpu.make_async_copy(
                xbufs[slot].at[b], out_hbm.at[b * _NB + p0], stsems[slot]))
        for d in descs:
            d.start()
        return descs

    loads = {}
    stores = {}
    loads[0] = issue_loads(0, 0)
    for c in range(_PB):
        slot = c % _DEPTH
        if c + 1 < _PB:
            nslot = (c + 1) % _DEPTH
            if c - _DEPTH + 1 >= 0:
                for d in stores[c - _DEPTH + 1]:
                    d.wait()
            loads[c + 1] = issue_loads(c + 1, nslot)
        for d in loads[c]:
            d.wait()
        _add_chunk(xbufs[slot], pbufs[slot])
        stores[c] = issue_stores(c, slot)
    for c in range(max(0, _PB - _DEPTH), _PB):
        for d in stores[c]:
            d.wait()


def kernel(x, pos_table):
    batch, seq_len, d_model = x.shape
    xv = x.reshape(batch * seq_len // _RB, _RB, d_model)
    pv = pos_table.reshape(seq_len // _RB, _RB, d_model)
    run = functools.partial(
        pl.kernel,
        out_type=jax.ShapeDtypeStruct(xv.shape, x.dtype),
        scratch_types=(
            [pltpu.VMEM((_RB, _D), jnp.float32)] * _DEPTH
            + [pltpu.VMEM((_B, _RB, _D), jnp.float32)] * _DEPTH
            + [pltpu.SemaphoreType.DMA] * (2 * _DEPTH)
        ),
        mesh=plsc.VectorSubcoreMesh(core_axis_name="c", subcore_axis_name="s"),
    )(_sc_body)
    out = run(xv, pv)
    return out.reshape(batch, seq_len, d_model)


# speedup vs baseline: 1.0538x; 1.0122x over previous
import functools

import jax
import jax.numpy as jnp
from jax import lax
from jax.experimental import pallas as pl
from jax.experimental.pallas import tpu as pltpu
from jax.experimental.pallas import tpu_sc as plsc

_B, _S, _D = 4, 8192, 1024
_NC, _NS = 2, 16
_NW = _NC * _NS
_RB = 8
_NB = _S // _RB
_PB = _NB // _NW
_GPR = _D // 16
_LANES = 16
_DEPTH = 3


_UNROLL = 8
_SUBR = _GPR // _UNROLL


def _add_chunk(xbuf, pbuf):
    def body(i, carry):
        r = i // _SUBR
        j = (i % _SUBR) * _UNROLL
        for u in range(_UNROLL):
            c16 = (j + u) * _LANES
            pv = pbuf[r, pl.ds(c16, _LANES)]
            for b in range(_B):
                xbuf[b, r, pl.ds(c16, _LANES)] = (
                    xbuf[b, r, pl.ds(c16, _LANES)] + pv)
        return carry
    lax.fori_loop(0, _RB * _SUBR, body, 0)


def _sc_body(x_hbm, pos_hbm, out_hbm, *scratch):
    pbufs = scratch[0:_DEPTH]
    xbufs = scratch[_DEPTH:2 * _DEPTH]
    ldsems = scratch[2 * _DEPTH:3 * _DEPTH]
    stsems = scratch[3 * _DEPTH:4 * _DEPTH]
    wid = lax.axis_index("s") * _NC + lax.axis_index("c")
    band_base = wid * _PB

    def issue_loads(c, slot):
        p0 = band_base + c
        descs = [
            pltpu.make_async_copy(
                pos_hbm.at[p0], pbufs[slot], ldsems[slot]),
            pltpu.make_async_copy(
                x_hbm.at[:, p0], xbufs[slot], ldsems[slot]),
        ]
        for d in descs:
            d.start()
        return descs

    def issue_stores(c, slot):
        p0 = band_base + c
        descs = [pltpu.make_async_copy(
            xbufs[slot], out_hbm.at[:, p0], stsems[slot])]
        for d in descs:
            d.start()
        return descs

    loads = {}
    stores = {}
    loads[0] = issue_loads(0, 0)
    for c in range(_PB):
        slot = c % _DEPTH
        if c + 1 < _PB:
            nslot = (c + 1) % _DEPTH
            if c - _DEPTH + 1 >= 0:
                for d in stores[c - _DEPTH + 1]:
                    d.wait()
            loads[c + 1] = issue_loads(c + 1, nslot)
        for d in loads[c]:
            d.wait()
        _add_chunk(xbufs[slot], pbufs[slot])
        stores[c] = issue_stores(c, slot)
    for c in range(max(0, _PB - _DEPTH), _PB):
        for d in stores[c]:
            d.wait()


def kernel(x, pos_table):
    batch, seq_len, d_model = x.shape
    xv = x.reshape(batch, seq_len // _RB, _RB, d_model)
    pv = pos_table.reshape(seq_len // _RB, _RB, d_model)
    run = functools.partial(
        pl.kernel,
        out_type=jax.ShapeDtypeStruct(xv.shape, x.dtype),
        scratch_types=(
            [pltpu.VMEM((_RB, _D), jnp.float32)] * _DEPTH
            + [pltpu.VMEM((_B, _RB, _D), jnp.float32)] * _DEPTH
            + [pltpu.SemaphoreType.DMA] * (2 * _DEPTH)
        ),
        mesh=plsc.VectorSubcoreMesh(core_axis_name="c", subcore_axis_name="s"),
    )(_sc_body)
    out = run(xv, pv)
    return out.reshape(batch, seq_len, d_model)


# speedup vs baseline: 1.5490x; 1.4699x over previous
import jax
import jax.numpy as jnp
from jax.experimental import pallas as pl
from jax.experimental.pallas import tpu as pltpu

_BLK = 2048


def _add_block(x_ref, p_ref, o_ref):
    o_ref[...] = x_ref[...] + p_ref[...]


def kernel(x, pos_table):
    batch, seq_len, d_model = x.shape
    nblk = seq_len // _BLK
    return pl.pallas_call(
        _add_block,
        grid=(nblk, batch),
        in_specs=[
            pl.BlockSpec((1, _BLK, d_model), lambda s, b: (b, s, 0)),
            pl.BlockSpec((_BLK, d_model), lambda s, b: (s, 0)),
        ],
        out_specs=pl.BlockSpec((1, _BLK, d_model), lambda s, b: (b, s, 0)),
        out_shape=jax.ShapeDtypeStruct(x.shape, x.dtype),
        compiler_params=pltpu.CompilerParams(
            dimension_semantics=("parallel", "parallel"),
        ),
    )(x, pos_table)
